# baseline (speedup 1.0000x reference)
import functools

import jax
import jax.numpy as jnp
from jax import lax
from jax.experimental import pallas as pl
from jax.experimental.pallas import tpu as pltpu
from jax.experimental.pallas import tpu_sc as plsc

D = 64
NC, NS, L = 2, 16, 16
NW = NC * NS
B = 4096
H = 200
G = 256
NBP = B // G
NGRP = H * NBP
PER_W = NGRP // NW
NV = G // L

_mesh = plsc.VectorSubcoreMesh(core_axis_name="c", subcore_axis_name="s")


@functools.partial(
    pl.kernel,
    out_type=jax.ShapeDtypeStruct((H, D // 8, B // 128, 8, 128), jnp.float32),
    mesh=_mesh,
    scratch_types=[
        pltpu.VMEM((2, G), jnp.int32),
        pltpu.VMEM((2, G), jnp.int32),
        pltpu.VMEM((2, G), jnp.int32),
        pltpu.VMEM((G,), jnp.int32),
        pltpu.VMEM((2, G, 128), jnp.float32),
        pltpu.VMEM((2, 8, 2, 8, 128), jnp.float32),
        pltpu.SemaphoreType.DMA((2,)),
        pltpu.SemaphoreType.DMA((2,)),
        pltpu.SemaphoreType.DMA((2,)),
    ],
    compiler_params=pltpu.CompilerParams(
        use_tc_tiling_on_sc=False, needs_layout_passes=False),
)
def _emb_lookup(xf_hbm, wv_hbm, out_hbm, idx_v, idxf_v, base_v, r128_v,
                rows_v, outt_v, s_idx, s_g, s_o):
    wid = lax.axis_index("s") * NC + lax.axis_index("c")
    j0 = wid * PER_W

    for k in range(NV):
        r128_v[pl.ds(k * L, L)] = lax.iota(jnp.int32, L) + (k * L)

    def q_of(j):
        g2 = j0 + j
        h = lax.shift_right_logical(g2, 4)
        btp = lax.bitwise_and(g2, NBP - 1)
        return h, btp, h * B + btp * G

    def start_idx(p, j):
        _, _, q0 = q_of(j)
        pltpu.async_copy(xf_hbm.at[pl.ds(q0, G)], idx_v.at[p], s_idx.at[p])

    def wait_idx(p):
        pltpu.make_async_copy(
            xf_hbm.at[pl.ds(0, G)], idx_v.at[p], s_idx.at[p]).wait()

    def fuse(p):
        for k in range(NV):
            v = idx_v[p, pl.ds(k * L, L)]
            idxf_v[p, pl.ds(k * L, L)] = lax.shift_right_logical(v, 1)
            base_v[p, pl.ds(k * L, L)] = lax.shift_left(
                lax.bitwise_and(v, 1), 6)

    def start_gather(p):
        pltpu.async_copy(wv_hbm.at[idxf_v.at[p]], rows_v.at[p], s_g.at[p])

    def wait_gather(p):
        pltpu.make_async_copy(
            wv_hbm.at[idxf_v.at[p]], rows_v.at[p], s_g.at[p]).wait()

    def transpose(p):
        @pl.loop(0, D // 8, unroll=4)
        def _dt(dt):
            d8 = dt * 8
            for k in range(NV):
                row = r128_v[pl.ds(k * L, L)]
                cb = base_v[p, pl.ds(k * L, L)]
                btl, k2 = k // 8, k % 8
                for dp in range(8):
                    outt_v[p, dt, btl, dp, pl.ds(k2 * L, L)] = (
                        plsc.load_gather(
                            rows_v.at[p], [row, cb + (d8 + dp)]))

    def start_write(p, j):
        h, btp, _ = q_of(j)
        pltpu.async_copy(
            outt_v.at[p], out_hbm.at[h, :, pl.ds(btp * 2, 2)], s_o.at[p])

    def wait_write(p, j):
        h, btp, _ = q_of(j)
        pltpu.make_async_copy(
            outt_v.at[p], out_hbm.at[h, :, pl.ds(btp * 2, 2)], s_o.at[p]
        ).wait()

    start_idx(0, 0)
    wait_idx(0)
    fuse(0)
    start_gather(0)
    start_idx(1, 1)

    @pl.loop(0, PER_W // 2)
    def _grp(t):
        for r in range(2):
            j = t * 2 + r
            p, pn = r, 1 - r

            @pl.when(j < PER_W - 1)
            def _():
                wait_idx(pn)
                fuse(pn)
                start_gather(pn)

            @pl.when(j < PER_W - 2)
            def _():
                start_idx(p, j + 2)

            wait_gather(p)

            @pl.when(j >= 2)
            def _():
                wait_write(p, j - 2)

            transpose(p)
            start_write(p, j)

    wait_write(0, PER_W - 2)
    wait_write(1, PER_W - 1)


def kernel(x, weight):
    xf = x.T.reshape(B * H)
    wv = weight.reshape(500000, 128)
    out5 = _emb_lookup(xf, wv)
    return out5.transpose((2, 4, 0, 1, 3)).reshape(B, H, D)


# speedup vs baseline: 1.6521x; 1.6521x over previous
import functools

import jax
import jax.numpy as jnp
from jax import lax
from jax.experimental import pallas as pl
from jax.experimental.pallas import tpu as pltpu
from jax.experimental.pallas import tpu_sc as plsc

D = 64
NC, NS = 2, 16
NW = NC * NS
B_TOTAL = 4096 * 200
PER_W = B_TOTAL // NW
CHUNK = 400
STEPS = PER_W // CHUNK
R = 4
GROUPS = (STEPS - 2 * R) // R

_mesh = plsc.VectorSubcoreMesh(core_axis_name="c", subcore_axis_name="s")


@functools.partial(
    pl.kernel,
    out_type=jax.ShapeDtypeStruct((B_TOTAL, D), jnp.float32),
    mesh=_mesh,
    scratch_types=[
        pltpu.VMEM((R, CHUNK), jnp.int32),
        pltpu.VMEM((R, CHUNK, D), jnp.float32),
        pltpu.SemaphoreType.DMA((R,)),
        pltpu.SemaphoreType.DMA((R,)),
        pltpu.SemaphoreType.DMA((R,)),
    ],
    compiler_params=pltpu.CompilerParams(use_tc_tiling_on_sc=False),
)
def _emb_lookup(x_hbm, w_hbm, out_hbm, idx_v, rows_v, si, sg, so):
    wid = lax.axis_index("s") * NC + lax.axis_index("c")
    base = wid * PER_W

    def start_idx(b, chunk):
        off = base + chunk * CHUNK
        pltpu.async_copy(x_hbm.at[pl.ds(off, CHUNK)], idx_v.at[b], si.at[b])

    def wait_idx(b):
        pltpu.make_async_copy(
            x_hbm.at[pl.ds(base, CHUNK)], idx_v.at[b], si.at[b]).wait()

    def start_gather(b):
        pltpu.async_copy(w_hbm.at[idx_v.at[b]], rows_v.at[b], sg.at[b])

    def wait_gather(b):
        pltpu.make_async_copy(
            w_hbm.at[idx_v.at[b]], rows_v.at[b], sg.at[b]).wait()

    def start_out(b, chunk):
        off = base + chunk * CHUNK
        pltpu.async_copy(rows_v.at[b], out_hbm.at[pl.ds(off, CHUNK)], so.at[b])

    def wait_out(b):
        pltpu.make_async_copy(
            rows_v.at[b], out_hbm.at[pl.ds(base, CHUNK)], so.at[b]).wait()

    for b in range(R):
        start_idx(b, b)
    for b in range(R):
        wait_idx(b)
        start_gather(b)
    for b in range(R - 1):
        wait_gather(b)
        start_out(b, b)
        start_idx(b, b + R)

    @pl.loop(0, GROUPS)
    def _grp(g):
        for r in range(R):
            i = R + g * R + r
            b = r
            bp = (r + R - 1) % R
            wait_gather(bp)
            start_out(bp, i - 1)
            start_idx(bp, i - 1 + R)
            wait_idx(b)
            wait_out(b)
            start_gather(b)

    for i in range(STEPS - R, STEPS):
        b = i % R
        bp = (i - 1) % R
        wait_gather(bp)
        start_out(bp, i - 1)
        if i - 1 + R < STEPS:
            start_idx(bp, i - 1 + R)
        wait_idx(b)
        wait_out(b)
        start_gather(b)
    b_last = (STEPS - 1) % R
    wait_gather(b_last)
    start_out(b_last, STEPS - 1)
    for b in range(R):
        wait_out(b)


def kernel(x, weight):
    B, H = x.shape
    flat = x.reshape(B * H)
    out = _emb_lookup(flat, weight)
    return out.reshape(B, H, D)
